# baseline (speedup 1.0000x reference)
import functools

import jax
import jax.numpy as jnp
from jax import lax
from jax.experimental import pallas as pl
from jax.experimental.pallas import tpu as pltpu
from jax.experimental.pallas import tpu_sc as plsc

B = 16384
NW = 32
BPW = B // NW
CH = 128
NCH = BPW // CH
BLK = 2048


def _sc_gather(user2d, item2d, gu_t, gi_t, mu_t, mi_t):
    mesh = plsc.VectorSubcoreMesh(core_axis_name="c", subcore_axis_name="s")

    @functools.partial(
        pl.kernel,
        mesh=mesh,
        compiler_params=pltpu.CompilerParams(use_tc_tiling_on_sc=False),
        out_type=[
            jax.ShapeDtypeStruct((B, 8), jnp.float32),
            jax.ShapeDtypeStruct((B, 8), jnp.float32),
            jax.ShapeDtypeStruct((B, 16), jnp.float32),
            jax.ShapeDtypeStruct((B, 16), jnp.float32),
        ],
        scratch_types=[
            pltpu.VMEM((NCH, CH), jnp.int32),
            pltpu.VMEM((NCH, CH), jnp.int32),
            pltpu.VMEM((BPW, 8), jnp.float32),
            pltpu.VMEM((BPW, 8), jnp.float32),
            pltpu.VMEM((BPW, 16), jnp.float32),
            pltpu.VMEM((BPW, 16), jnp.float32),
            pltpu.SemaphoreType.DMA,
        ],
    )
    def k(user_h, item_h, gu_h, gi_h, mu_h, mi_h,
          gu_o, gi_o, mu_o, mi_o,
          uidx, iidx, bgu, bgi, bmu, bmi, sem):
        wid = lax.axis_index("s") * 2 + lax.axis_index("c")
        base = wid * BPW
        pltpu.sync_copy(user_h.at[pl.ds(wid * NCH, NCH)], uidx)
        pltpu.sync_copy(item_h.at[pl.ds(wid * NCH, NCH)], iidx)
        copies = []
        for j in range(NCH):
            s = pl.ds(j * CH, CH)
            copies.append(pltpu.async_copy(gu_h.at[uidx.at[j]], bgu.at[s], sem))
            copies.append(pltpu.async_copy(gi_h.at[iidx.at[j]], bgi.at[s], sem))
            copies.append(pltpu.async_copy(mu_h.at[uidx.at[j]], bmu.at[s], sem))
            copies.append(pltpu.async_copy(mi_h.at[iidx.at[j]], bmi.at[s], sem))
        for c in copies:
            c.wait()
        pltpu.sync_copy(bgu, gu_o.at[pl.ds(base, BPW)])
        pltpu.sync_copy(bgi, gi_o.at[pl.ds(base, BPW)])
        pltpu.sync_copy(bmu, mu_o.at[pl.ds(base, BPW)])
        pltpu.sync_copy(bmi, mi_o.at[pl.ds(base, BPW)])

    return k(user2d, item2d, gu_t, gi_t, mu_t, mi_t)


def _tc_body(gu, gi, mu, mi, W1, b1, W2, b2, Wp, bp, out):
    w1 = W1[...]
    h = jnp.maximum(
        jnp.dot(mu[...], w1[:16, :], preferred_element_type=jnp.float32)
        + jnp.dot(mi[...], w1[16:, :], preferred_element_type=jnp.float32)
        + b1[...],
        0.0,
    )
    m = jnp.maximum(
        jnp.dot(h, W2[...], preferred_element_type=jnp.float32) + b2[...], 0.0
    )
    g = gu[...] * gi[...]
    wp = Wp[...]
    s = (
        jnp.dot(g, wp[:8, :], preferred_element_type=jnp.float32)
        + jnp.dot(m, wp[8:, :], preferred_element_type=jnp.float32)
        + bp[...]
    )
    out[...] = jax.nn.sigmoid(s)


def _tc_dense(gu_r, gi_r, mu_r, mi_r, W1, b1, W2, b2, Wp, bp):
    grid = B // BLK
    return pl.pallas_call(
        _tc_body,
        grid=(grid,),
        in_specs=[
            pl.BlockSpec((BLK, 8), lambda i: (i, 0)),
            pl.BlockSpec((BLK, 8), lambda i: (i, 0)),
            pl.BlockSpec((BLK, 16), lambda i: (i, 0)),
            pl.BlockSpec((BLK, 16), lambda i: (i, 0)),
            pl.BlockSpec((32, 16), lambda i: (0, 0)),
            pl.BlockSpec((1, 16), lambda i: (0, 0)),
            pl.BlockSpec((16, 8), lambda i: (0, 0)),
            pl.BlockSpec((1, 8), lambda i: (0, 0)),
            pl.BlockSpec((16, 1), lambda i: (0, 0)),
            pl.BlockSpec((1, 1), lambda i: (0, 0)),
        ],
        out_specs=pl.BlockSpec((BLK, 1), lambda i: (i, 0)),
        out_shape=jax.ShapeDtypeStruct((B, 1), jnp.float32),
    )(gu_r, gi_r, mu_r, mi_r, W1, b1, W2, b2, Wp, bp)


def kernel(user, item, gmf_user_emb, gmf_item_emb, mlp_user_emb, mlp_item_emb,
           W1, b1, W2, b2, Wp, bp):
    user2d = user.astype(jnp.int32).reshape(NW * NCH, CH)
    item2d = item.astype(jnp.int32).reshape(NW * NCH, CH)
    gu_r, gi_r, mu_r, mi_r = _sc_gather(
        user2d, item2d, gmf_user_emb, gmf_item_emb, mlp_user_emb, mlp_item_emb)
    out = _tc_dense(
        gu_r, gi_r, mu_r, mi_r,
        W1, b1.reshape(1, 16), W2, b2.reshape(1, 8), Wp, bp.reshape(1, 1))
    return out.reshape(-1)
